# baseline (speedup 1.0000x reference)
import jax
import jax.numpy as jnp
from jax.experimental import pallas as pl
from jax.experimental.pallas import tpu as pltpu


def _cast_kernel(m_ref, o_ref):
    o_ref[...] = m_ref[...].astype(jnp.bfloat16)


def _matmul_kernel(x_ref, m_ref, o_ref):
    x = x_ref[...].astype(jnp.bfloat16)
    n2 = m_ref.shape[1] // 2
    o_ref[:, :n2] = jnp.dot(x, m_ref[:, :n2],
                            preferred_element_type=jnp.float32)
    o_ref[:, n2:] = jnp.dot(x, m_ref[:, n2:],
                            preferred_element_type=jnp.float32)


def kernel(x, M):
    B, C, H, W = x.shape
    K = H * W
    N = M.shape[1]
    Ho = 2 * H
    Wo = N // Ho

    xf = x.reshape(B * C, K)

    Mb = pl.pallas_call(
        _cast_kernel,
        out_shape=jax.ShapeDtypeStruct((K, N), jnp.bfloat16),
        in_specs=[pl.BlockSpec((K, N), lambda: (0, 0))],
        out_specs=pl.BlockSpec((K, N), lambda: (0, 0)),
    )(M)

    BM = 256
    yf = pl.pallas_call(
        _matmul_kernel,
        out_shape=jax.ShapeDtypeStruct((B * C, N), jnp.float32),
        grid=(B * C // BM,),
        in_specs=[
            pl.BlockSpec((BM, K), lambda i: (i, 0)),
            pl.BlockSpec((K, N), lambda i: (0, 0)),
        ],
        out_specs=pl.BlockSpec((BM, N), lambda i: (i, 0)),
        compiler_params=pltpu.CompilerParams(
            dimension_semantics=("parallel",)),
    )(xf, Mb)
    return yf.reshape(B, C, Ho, Wo)


# speedup vs baseline: 2.3238x; 2.3238x over previous
import jax
import jax.numpy as jnp
from jax.experimental import pallas as pl
from jax.experimental.pallas import tpu as pltpu


def _matmul_kernel(x_ref, m_ref, o_ref):
    x = x_ref[0]
    n2 = m_ref.shape[1] // 2
    o_ref[0, :, :n2] = jnp.dot(x, m_ref[:, :n2],
                               preferred_element_type=jnp.float32)
    o_ref[0, :, n2:] = jnp.dot(x, m_ref[:, n2:],
                               preferred_element_type=jnp.float32)


def kernel(x, M):
    B, C, H, W = x.shape
    K = H * W
    N = M.shape[1]
    Ho = 2 * H
    Wo = N // Ho

    xf = x.reshape(B, C, K)
    yf = pl.pallas_call(
        _matmul_kernel,
        out_shape=jax.ShapeDtypeStruct((B, C, N), jnp.float32),
        grid=(B,),
        in_specs=[
            pl.BlockSpec((1, C, K), lambda b: (b, 0, 0)),
            pl.BlockSpec((K, N), lambda b: (0, 0)),
        ],
        out_specs=pl.BlockSpec((1, C, N), lambda b: (b, 0, 0)),
        compiler_params=pltpu.CompilerParams(
            dimension_semantics=("parallel",)),
    )(xf, M)
    return yf.reshape(B, C, Ho, Wo)
